# baseline (speedup 1.0000x reference)
import functools

import jax
import jax.numpy as jnp
from jax import lax
from jax.experimental import pallas as pl
from jax.experimental.pallas import tpu as pltpu
from jax.experimental.pallas import tpu_sc as plsc

BATCH = 512
NUM_NEURONS = 8192
HISTORY_LEN = 8
_NL = 128
_NC = NUM_NEURONS // _NL
_NW = 32
_RPW = BATCH // _NW


def kernel(x, state):
    st4 = state.reshape(BATCH, _NC, _NL, HISTORY_LEN).transpose(0, 1, 3, 2)
    x4 = x.reshape(BATCH // 8, 8, _NC, _NL).transpose(0, 2, 1, 3)

    mesh = plsc.VectorSubcoreMesh(core_axis_name="c", subcore_axis_name="s")

    @functools.partial(
        pl.kernel,
        mesh=mesh,
        out_type=jax.ShapeDtypeStruct((BATCH, _NC, HISTORY_LEN, _NL), jnp.float32),
        scratch_types=[
            pltpu.VMEM((_NC, HISTORY_LEN - 1, _NL), jnp.float32),
            pltpu.VMEM((_NC, _NL), jnp.float32),
        ],
    )
    def push(x4_hbm, st4_hbm, out4_hbm, stbuf, xbuf):
        wid = lax.axis_index("s") * 2 + lax.axis_index("c")
        base = wid * _RPW

        def body(i, carry):
            b = base + i
            pltpu.sync_copy(st4_hbm.at[b, :, 0 : HISTORY_LEN - 1, :], stbuf)
            pltpu.sync_copy(stbuf, out4_hbm.at[b, :, 1:HISTORY_LEN, :])
            pltpu.sync_copy(x4_hbm.at[b // 8, :, b % 8, :], xbuf)
            pltpu.sync_copy(xbuf, out4_hbm.at[b, :, 0, :])
            return carry

        lax.fori_loop(0, _RPW, body, 0)

    out4 = push(x4, st4)
    return out4.transpose(0, 1, 3, 2).reshape(BATCH, NUM_NEURONS, HISTORY_LEN)


# speedup vs baseline: 1.1237x; 1.1237x over previous
import functools

import jax
import jax.numpy as jnp
from jax import lax
from jax.experimental import pallas as pl
from jax.experimental.pallas import tpu as pltpu
from jax.experimental.pallas import tpu_sc as plsc

BATCH = 512
NUM_NEURONS = 8192
HISTORY_LEN = 8
_NL = 128
_NC = NUM_NEURONS // _NL
_NW = 32
_RPW = BATCH // _NW


def kernel(x, state):
    st4 = state.reshape(BATCH, _NC, _NL, HISTORY_LEN).transpose(0, 1, 3, 2)
    x4 = x.reshape(BATCH // 8, 8, _NC, _NL).transpose(0, 2, 1, 3)

    mesh = plsc.VectorSubcoreMesh(core_axis_name="c", subcore_axis_name="s")

    _HC = _NC // 2

    @functools.partial(
        pl.kernel,
        mesh=mesh,
        out_type=jax.ShapeDtypeStruct((BATCH, _NC, HISTORY_LEN, _NL), jnp.float32),
        scratch_types=[
            pltpu.VMEM((2, _HC, HISTORY_LEN - 1, _NL), jnp.float32),
            pltpu.VMEM((2, _HC, _NL), jnp.float32),
            pltpu.SemaphoreType.DMA((2,)),
            pltpu.SemaphoreType.DMA((2,)),
            pltpu.SemaphoreType.DMA((2,)),
            pltpu.SemaphoreType.DMA((2,)),
        ],
    )
    def push(x4_hbm, st4_hbm, out4_hbm, stbuf, xbuf, sgs, sgx, sss, ssx):
        wid = lax.axis_index("s") * 2 + lax.axis_index("c")
        base = wid * _RPW

        pend = [None, None]
        for u in range(_RPW * 2):
            b = base + u // 2
            c0 = (u % 2) * _HC
            j = u & 1
            if pend[j] is not None:
                pend[j][0].wait()
                pend[j][1].wait()
            g1 = pltpu.async_copy(
                st4_hbm.at[b, c0 : c0 + _HC, 0 : HISTORY_LEN - 1, :],
                stbuf.at[j],
                sgs.at[j],
            )
            g2 = pltpu.async_copy(
                x4_hbm.at[b // 8, c0 : c0 + _HC, b % 8, :], xbuf.at[j], sgx.at[j]
            )
            g1.wait()
            g2.wait()
            s1 = pltpu.async_copy(
                stbuf.at[j],
                out4_hbm.at[b, c0 : c0 + _HC, 1:HISTORY_LEN, :],
                sss.at[j],
            )
            s2 = pltpu.async_copy(
                xbuf.at[j], out4_hbm.at[b, c0 : c0 + _HC, 0, :], ssx.at[j]
            )
            pend[j] = (s1, s2)
        for j in range(2):
            pend[j][0].wait()
            pend[j][1].wait()

    out4 = push(x4, st4)
    return out4.transpose(0, 1, 3, 2).reshape(BATCH, NUM_NEURONS, HISTORY_LEN)
